# baseline (speedup 1.0000x reference)
import functools

import jax
import jax.numpy as jnp
from jax import lax
from jax.experimental import pallas as pl
from jax.experimental.pallas import tpu as pltpu
from jax.experimental.pallas import tpu_sc as plsc

VOCAB = 1000000
EMBED = 32
BATCH = 16384
HIST = 50

NC = 2
NS = 16
NW = NC * NS

BB = 128
NBLK = BATCH // BB
BLK_PER_W = NBLK // NW
DSTW = 129


def _make_kernel():
  mesh = plsc.VectorSubcoreMesh(core_axis_name="c", subcore_axis_name="s")

  @functools.partial(
      pl.kernel,
      out_type=jax.ShapeDtypeStruct((HIST, EMBED, BATCH), jnp.float32),
      mesh=mesh,
      compiler_params=pltpu.CompilerParams(needs_layout_passes=False),
      scratch_types=[
          pltpu.VMEM((56, BB), jnp.int32),
          pltpu.VMEM((BB,), jnp.int32),
          pltpu.VMEM((BB,), jnp.int32),
          pltpu.VMEM((56, BB), jnp.int32),
          pltpu.VMEM((BB, DSTW), jnp.float32),
          pltpu.VMEM((BB, DSTW), jnp.float32),
          pltpu.VMEM((EMBED, BB), jnp.float32),
          pltpu.VMEM((EMBED, BB), jnp.float32),
          pltpu.SemaphoreType.DMA,
          pltpu.SemaphoreType.DMA,
          pltpu.SemaphoreType.DMA,
          pltpu.SemaphoreType.DMA,
      ],
  )
  def gather_kernel(xt_hbm, t2_hbm, out_hbm, xv, qv0, qv1, cv,
                    dst0, dst1, panel0, panel1, sg0, sg1, sp0, sp1):
    qvs = (qv0, qv1)
    dsts = (dst0, dst1)
    panels = (panel0, panel1)
    sgs = (sg0, sg1)
    sps = (sp0, sp1)
    wid = lax.axis_index("s") * NC + lax.axis_index("c")
    iota = lax.iota(jnp.int32, 16)

    @pl.loop(0, BLK_PER_W)
    def _blk(blk):
      b0 = (wid * BLK_PER_W + blk) * BB
      for h0 in range(0, 48, 8):
        pltpu.sync_copy(xt_hbm.at[pl.ds(h0, 8), pl.ds(b0, BB)],
                        xv.at[pl.ds(h0, 8)])
      pltpu.sync_copy(xt_hbm.at[pl.ds(48, 2), pl.ds(b0, BB)],
                      xv.at[pl.ds(48, 2)])

      def line(h, slot):
        qv = qvs[slot]
        for g in range(8):
          v = xv[h, pl.ds(g * 16, 16)]
          qv[pl.ds(g * 16, 16)] = lax.shift_right_logical(v, 2)
          cv[h, pl.ds(g * 16, 16)] = lax.shift_left(jnp.bitwise_and(v, 3), 5)
        pltpu.async_copy(t2_hbm.at[qvs[slot]],
                         dsts[slot].at[:, pl.ds(0, 128)], sgs[slot])

      def select(h, slot):
        pltpu.make_async_copy(t2_hbm.at[qvs[slot]],
                              dsts[slot].at[:, pl.ds(0, 128)],
                              sgs[slot]).wait()
        dstp = dsts[slot]
        pan = panels[slot]
        for g in range(8):
          rows = g * 16 + iota
          cvec = cv[h, pl.ds(g * 16, 16)]
          for e in range(EMBED):
            val = plsc.load_gather(dstp, [rows, cvec + e])
            pan[e, pl.ds(g * 16, 16)] = val
        pltpu.async_copy(pan, out_hbm.at[h, :, pl.ds(b0, BB)], sps[slot])

      @pl.loop(0, HIST // 2)
      def _hpair(i):
        h0 = 2 * i
        h1 = 2 * i + 1
        line(h0, 0)
        line(h1, 1)
        select(h0, 0)
        select(h1, 1)
        pltpu.make_async_copy(
            panels[0], out_hbm.at[h0, :, pl.ds(b0, BB)], sps[0]).wait()
        pltpu.make_async_copy(
            panels[1], out_hbm.at[h1, :, pl.ds(b0, BB)], sps[1]).wait()

  return gather_kernel


_gather = _make_kernel()


@jax.jit
def kernel(x, table):
  xt = jnp.transpose(x)
  t2 = (jnp.transpose(table)
        .reshape(EMBED, VOCAB // 4, 4)
        .transpose(1, 2, 0)
        .reshape(VOCAB // 4, 128))
  out_t = _gather(xt, t2)
  return jnp.transpose(out_t, (2, 0, 1))
